# baseline (speedup 1.0000x reference)
import functools

import jax
import jax.numpy as jnp
from jax import lax
from jax.experimental import pallas as pl
from jax.experimental.pallas import tpu as pltpu
from jax.experimental.pallas import tpu_sc as plsc


LANES = 16


@functools.lru_cache(maxsize=None)
def _make_sc_pool(B, L, D):
    info = plsc.get_sparse_core_info()
    NC, NS = info.num_cores, info.num_subcores
    NW = NC * NS
    assert B % NW == 0
    RPW = B // NW

    CH0 = min(L, 128)
    CH0 -= CH0 % 8
    CH1 = L - CH0
    assert 0 < CH0 <= 128 and 0 <= CH1 <= 128 and CH0 % 8 == 0

    mesh = plsc.VectorSubcoreMesh(core_axis_name="c", subcore_axis_name="s")

    @functools.partial(
        pl.kernel,
        mesh=mesh,
        compiler_params=pltpu.CompilerParams(use_tc_tiling_on_sc=False),
        out_type=jax.ShapeDtypeStruct((B, D), jnp.float32),
        scratch_types=[
            pltpu.VMEM((RPW, L), jnp.int32),
            pltpu.VMEM((L, D), jnp.float32),
            pltpu.VMEM((RPW, D), jnp.float32),
            pltpu.SemaphoreType.DMA,
        ],
    )
    def sc_pool(x_hbm, emb_hbm, out_hbm, idx_v, rows_v, out_v, sem):
        wid = lax.axis_index("s") * NC + lax.axis_index("c")
        base = wid * RPW
        pltpu.sync_copy(x_hbm.at[pl.ds(base, RPW)], idx_v)

        nch = D // LANES

        def row_body(i, carry):
            cp0 = pltpu.async_copy(
                emb_hbm.at[idx_v.at[i, pl.ds(0, CH0)]],
                rows_v.at[pl.ds(0, CH0)], sem)
            if CH1:
                cp1 = pltpu.async_copy(
                    emb_hbm.at[idx_v.at[i, pl.ds(CH0, CH1)]],
                    rows_v.at[pl.ds(CH0, CH1)], sem)
            cp0.wait()
            if CH1:
                cp1.wait()

            accs = tuple(rows_v[0, pl.ds(c * LANES, LANES)] for c in range(nch))

            def max_body(r, accs):
                return tuple(
                    jnp.maximum(a, rows_v[r, pl.ds(c * LANES, LANES)])
                    for c, a in enumerate(accs))

            accs = lax.fori_loop(1, L, max_body, accs)
            for c in range(nch):
                out_v[i, pl.ds(c * LANES, LANES)] = accs[c]
            return carry

        lax.fori_loop(0, RPW, row_body, 0)
        pltpu.sync_copy(out_v, out_hbm.at[pl.ds(base, RPW)])

    return sc_pool


@functools.lru_cache(maxsize=None)
def _make_tc_linear(B, D, C):
    BLK = min(B, 512)
    assert B % BLK == 0

    def body(p_ref, w_ref, b_ref, o_ref):
        acc = jnp.dot(p_ref[...], w_ref[...], preferred_element_type=jnp.float32)
        o_ref[...] = jax.nn.sigmoid(acc + b_ref[...])

    return pl.pallas_call(
        body,
        grid=(B // BLK,),
        in_specs=[
            pl.BlockSpec((BLK, D), lambda i: (i, 0)),
            pl.BlockSpec((D, C), lambda i: (0, 0)),
            pl.BlockSpec((1, C), lambda i: (0, 0)),
        ],
        out_specs=pl.BlockSpec((BLK, C), lambda i: (i, 0)),
        out_shape=jax.ShapeDtypeStruct((B, C), jnp.float32),
    )


def kernel(x, emb, fc_w, fc_b):
    B, L = x.shape
    D = emb.shape[1]
    C = fc_w.shape[0]
    x = x.astype(jnp.int32)
    pooled = _make_sc_pool(B, L, D)(x, emb)
    out = _make_tc_linear(B, D, C)(pooled, fc_w.T, fc_b.reshape(1, C))
    return out


# speedup vs baseline: 1.2075x; 1.2075x over previous
import functools

import jax
import jax.numpy as jnp
from jax import lax
from jax.experimental import pallas as pl
from jax.experimental.pallas import tpu as pltpu
from jax.experimental.pallas import tpu_sc as plsc


LANES = 16
GRP = 2


@functools.lru_cache(maxsize=None)
def _make_sc_pool(B, L, D):
    info = plsc.get_sparse_core_info()
    NC, NS = info.num_cores, info.num_subcores
    NW = NC * NS
    assert B % NW == 0
    RPW = B // NW
    assert RPW % (2 * GRP) == 0
    NG = RPW // GRP
    nch = D // LANES

    CH0 = min(L, 128)
    CH0 -= CH0 % 8
    CH1 = L - CH0
    assert 0 < CH0 <= 128 and 0 <= CH1 <= 128 and CH0 % 8 == 0

    mesh = plsc.VectorSubcoreMesh(core_axis_name="c", subcore_axis_name="s")

    @functools.partial(
        pl.kernel,
        mesh=mesh,
        compiler_params=pltpu.CompilerParams(use_tc_tiling_on_sc=False),
        out_type=jax.ShapeDtypeStruct((B, D), jnp.float32),
        scratch_types=[
            pltpu.VMEM((RPW, L), jnp.int32),
            pltpu.VMEM((2 * GRP, L, D), jnp.float32),
            pltpu.VMEM((RPW, D), jnp.float32),
            pltpu.SemaphoreType.DMA,
            pltpu.SemaphoreType.DMA,
        ],
    )
    def sc_pool(x_hbm, emb_hbm, out_hbm, idx_v, rows_v, out_v, sem0, sem1):
        wid = lax.axis_index("s") * NC + lax.axis_index("c")
        base = wid * RPW
        pltpu.sync_copy(x_hbm.at[pl.ds(base, RPW)], idx_v)

        def fire(g, half, sem):
            for j in range(GRP):
                row = g * GRP + j
                slot = half * GRP + j
                pltpu.async_copy(
                    emb_hbm.at[idx_v.at[row, pl.ds(0, CH0)]],
                    rows_v.at[slot, pl.ds(0, CH0)], sem)
                if CH1:
                    pltpu.async_copy(
                        emb_hbm.at[idx_v.at[row, pl.ds(CH0, CH1)]],
                        rows_v.at[slot, pl.ds(CH0, CH1)], sem)

        def drain(g, half, sem):
            for j in range(GRP):
                row = g * GRP + j
                slot = half * GRP + j
                pltpu.make_async_copy(
                    emb_hbm.at[idx_v.at[row, pl.ds(0, CH0)]],
                    rows_v.at[slot, pl.ds(0, CH0)], sem).wait()
                if CH1:
                    pltpu.make_async_copy(
                        emb_hbm.at[idx_v.at[row, pl.ds(CH0, CH1)]],
                        rows_v.at[slot, pl.ds(CH0, CH1)], sem).wait()

        def compute(g, half):
            for j in range(GRP):
                row = g * GRP + j
                buf = rows_v.at[half * GRP + j]
                accs = tuple(buf[0, pl.ds(c * LANES, LANES)]
                             for c in range(nch))

                def max_body(r, accs):
                    return tuple(
                        jnp.maximum(a, buf[r, pl.ds(c * LANES, LANES)])
                        for c, a in enumerate(accs))

                accs = lax.fori_loop(1, L, max_body, accs, unroll=8)
                for c in range(nch):
                    out_v[row, pl.ds(c * LANES, LANES)] = accs[c]

        fire(0, 0, sem0)

        def body(h, carry):
            g = 2 * h
            fire(g + 1, 1, sem1)
            drain(g, 0, sem0)
            compute(g, 0)

            @pl.when(h < NG // 2 - 1)
            def _():
                fire(g + 2, 0, sem0)

            drain(g + 1, 1, sem1)
            compute(g + 1, 1)
            return carry

        lax.fori_loop(0, NG // 2, body, 0)
        pltpu.sync_copy(out_v, out_hbm.at[pl.ds(base, RPW)])

    return sc_pool


@functools.lru_cache(maxsize=None)
def _make_tc_linear(B, D, C):
    BLK = min(B, 512)
    assert B % BLK == 0

    def body(p_ref, w_ref, b_ref, o_ref):
        acc = jnp.dot(p_ref[...], w_ref[...], preferred_element_type=jnp.float32)
        o_ref[...] = jax.nn.sigmoid(acc + b_ref[...])

    return pl.pallas_call(
        body,
        grid=(B // BLK,),
        in_specs=[
            pl.BlockSpec((BLK, D), lambda i: (i, 0)),
            pl.BlockSpec((D, C), lambda i: (0, 0)),
            pl.BlockSpec((1, C), lambda i: (0, 0)),
        ],
        out_specs=pl.BlockSpec((BLK, C), lambda i: (i, 0)),
        out_shape=jax.ShapeDtypeStruct((B, C), jnp.float32),
    )


def kernel(x, emb, fc_w, fc_b):
    B, L = x.shape
    D = emb.shape[1]
    C = fc_w.shape[0]
    x = x.astype(jnp.int32)
    pooled = _make_sc_pool(B, L, D)(x, emb)
    out = _make_tc_linear(B, D, C)(pooled, fc_w.T, fc_b.reshape(1, C))
    return out


# speedup vs baseline: 1.9081x; 1.5802x over previous
import functools

import jax
import jax.numpy as jnp
from jax import lax
from jax.experimental import pallas as pl
from jax.experimental.pallas import tpu as pltpu
from jax.experimental.pallas import tpu_sc as plsc


LANES = 16
GRP = 2


def _split_point(V):
    BP = 2048
    nb = -(-((V + 1) // 2) // BP)
    return BP * nb, BP, nb


@functools.lru_cache(maxsize=None)
def _make_tc_relayout(V, D):
    H, BP, nb = _split_point(V)

    def body(a_ref, b_ref, o_ref):
        o_ref[:, 0:D] = a_ref[...].T
        o_ref[:, D:2 * D] = b_ref[...].T

    return pl.pallas_call(
        body,
        grid=(nb,),
        in_specs=[
            pl.BlockSpec((D, BP), lambda j: (0, j)),
            pl.BlockSpec(
                (D, BP),
                lambda j, off=nb, last=(V - 1) // BP:
                    (0, jnp.minimum(j + off, last)),
            ),
        ],
        out_specs=pl.BlockSpec((BP, 2 * D), lambda j: (j, 0)),
        out_shape=jax.ShapeDtypeStruct((H, 2 * D), jnp.float32),
    )


@functools.lru_cache(maxsize=None)
def _make_sc_pool(B, L, D, V):
    H = _split_point(V)[0]
    info = plsc.get_sparse_core_info()
    NC, NS = info.num_cores, info.num_subcores
    NW = NC * NS
    assert B % NW == 0
    RPW = B // NW
    assert RPW % (2 * GRP) == 0
    NG = RPW // GRP
    nch = D // LANES
    NIV = (RPW * L) // LANES

    CH0 = min(L, 128)
    CH0 -= CH0 % 8
    CH1 = L - CH0
    assert 0 < CH0 <= 128 and 0 <= CH1 <= 128 and CH0 % 8 == 0

    mesh = plsc.VectorSubcoreMesh(core_axis_name="c", subcore_axis_name="s")

    @functools.partial(
        pl.kernel,
        mesh=mesh,
        compiler_params=pltpu.CompilerParams(use_tc_tiling_on_sc=False),
        out_type=jax.ShapeDtypeStruct((B, D), jnp.float32),
        scratch_types=[
            pltpu.VMEM((RPW * L,), jnp.int32),
            pltpu.VMEM((2 * GRP, L, D), jnp.float32),
            pltpu.VMEM((RPW, D), jnp.float32),
            pltpu.SemaphoreType.DMA,
            pltpu.SemaphoreType.DMA,
        ],
    )
    def sc_pool(x_hbm, emb_hbm, out_hbm, idx_v, rows_v, out_v, sem0, sem1):
        wid = lax.axis_index("s") * NC + lax.axis_index("c")
        base = wid * RPW
        pltpu.sync_copy(x_hbm.at[pl.ds(base * L, RPW * L)], idx_v)

        def remap_body(i, carry):
            v = idx_v[pl.ds(i * LANES, LANES)]
            v2 = v + v
            vrow = jnp.where(v2 >= 2 * H, v2 - (2 * H - 1), v2)
            idx_v[pl.ds(i * LANES, LANES)] = vrow
            return carry

        lax.fori_loop(0, NIV, remap_body, 0, unroll=8)

        def fire(g, half, sem):
            for j in range(GRP):
                row = g * GRP + j
                slot = half * GRP + j
                pltpu.async_copy(
                    emb_hbm.at[idx_v.at[pl.ds(row * L, CH0)]],
                    rows_v.at[slot, pl.ds(0, CH0)], sem)
                if CH1:
                    pltpu.async_copy(
                        emb_hbm.at[idx_v.at[pl.ds(row * L + CH0, CH1)]],
                        rows_v.at[slot, pl.ds(CH0, CH1)], sem)

        def drain(g, half, sem):
            for j in range(GRP):
                row = g * GRP + j
                slot = half * GRP + j
                pltpu.make_async_copy(
                    emb_hbm.at[idx_v.at[pl.ds(row * L, CH0)]],
                    rows_v.at[slot, pl.ds(0, CH0)], sem).wait()
                if CH1:
                    pltpu.make_async_copy(
                        emb_hbm.at[idx_v.at[pl.ds(row * L + CH0, CH1)]],
                        rows_v.at[slot, pl.ds(CH0, CH1)], sem).wait()

        def compute(g, half):
            for j in range(GRP):
                row = g * GRP + j
                buf = rows_v.at[half * GRP + j]
                accs = tuple(buf[0, pl.ds(c * LANES, LANES)]
                             for c in range(nch))

                def max_body(r, accs):
                    return tuple(
                        jnp.maximum(a, buf[r, pl.ds(c * LANES, LANES)])
                        for c, a in enumerate(accs))

                accs = lax.fori_loop(1, L, max_body, accs, unroll=8)
                for c in range(nch):
                    out_v[row, pl.ds(c * LANES, LANES)] = accs[c]

        fire(0, 0, sem0)

        def body(h, carry):
            g = 2 * h
            fire(g + 1, 1, sem1)
            drain(g, 0, sem0)
            compute(g, 0)

            @pl.when(h < NG // 2 - 1)
            def _():
                fire(g + 2, 0, sem0)

            drain(g + 1, 1, sem1)
            compute(g + 1, 1)
            return carry

        lax.fori_loop(0, NG // 2, body, 0)
        pltpu.sync_copy(out_v, out_hbm.at[pl.ds(base, RPW)])

    return sc_pool


@functools.lru_cache(maxsize=None)
def _make_tc_linear(B, D, C):
    BLK = min(B, 512)
    assert B % BLK == 0

    def body(p_ref, w_ref, b_ref, o_ref):
        acc = jnp.dot(p_ref[...], w_ref[...], preferred_element_type=jnp.float32)
        o_ref[...] = jax.nn.sigmoid(acc + b_ref[...])

    return pl.pallas_call(
        body,
        grid=(B // BLK,),
        in_specs=[
            pl.BlockSpec((BLK, D), lambda i: (i, 0)),
            pl.BlockSpec((D, C), lambda i: (0, 0)),
            pl.BlockSpec((1, C), lambda i: (0, 0)),
        ],
        out_specs=pl.BlockSpec((BLK, C), lambda i: (i, 0)),
        out_shape=jax.ShapeDtypeStruct((B, C), jnp.float32),
    )


def kernel(x, emb, fc_w, fc_b):
    B, L = x.shape
    V, D = emb.shape
    C = fc_w.shape[0]
    x = x.astype(jnp.int32)
    embT = jnp.swapaxes(emb, 0, 1)
    tbl2d = _make_tc_relayout(V, D)(embT, embT)
    tbl = jnp.reshape(tbl2d, (tbl2d.shape[0] * 2, D))
    xflat = jnp.reshape(x, (B * L,))
    pooled = _make_sc_pool(B, L, D, V)(xflat, tbl)

    out = _make_tc_linear(B, D, C)(pooled, fc_w.T, fc_b.reshape(1, C))
    return out


# speedup vs baseline: 2.3976x; 1.2566x over previous
import functools

import jax
import numpy as np
import jax.numpy as jnp
from jax import lax
from jax.experimental import pallas as pl
from jax.experimental.pallas import tpu as pltpu
from jax.experimental.pallas import tpu_sc as plsc


LANES = 16
GRP = 2


def _split_point(V):
    BP = 2048
    nq = -(-((V + 3) // 4) // BP)
    return BP * nq, BP, nq


@functools.lru_cache(maxsize=None)
def _make_tc_relayout(V, D):
    Q, BP, nq = _split_point(V)
    last = (V - 1) // BP
    W = D // 2

    def body(a0, a1, a2, a3, o_ref):
        for k, a in enumerate((a0, a1, a2, a3)):
            u = jax.lax.bitcast_convert_type(a[...], jnp.uint32)
            rb = (u + jnp.uint32(0x7FFF) + ((u >> 16) & jnp.uint32(1))) >> 16
            lo = rb[0:W, :]
            hi = rb[W:2 * W, :]
            o_ref[:, k * W:(k + 1) * W] = (lo | (hi << 16)).T

    def imap(k):
        return lambda j: (0, jnp.minimum(j + k * nq, last))

    return pl.pallas_call(
        body,
        grid=(nq,),
        in_specs=[pl.BlockSpec((D, BP), imap(k)) for k in range(4)],
        out_specs=pl.BlockSpec((BP, 4 * W), lambda j: (j, 0)),
        out_shape=jax.ShapeDtypeStruct((Q, 4 * W), jnp.uint32),
    )


@functools.lru_cache(maxsize=None)
def _make_sc_pool(B, L, D, V):
    Q = _split_point(V)[0]
    W = D // 2
    info = plsc.get_sparse_core_info()
    NC, NS = info.num_cores, info.num_subcores
    NW = NC * NS
    assert B % NW == 0
    RPW = B // NW
    assert RPW % (2 * GRP) == 0
    NG = RPW // GRP
    nch = W // LANES
    NIV = (RPW * L) // LANES

    CH0 = min(L, 128)
    CH0 -= CH0 % 8
    CH1 = L - CH0
    assert 0 < CH0 <= 128 and 0 <= CH1 <= 128 and CH0 % 8 == 0

    mesh = plsc.VectorSubcoreMesh(core_axis_name="c", subcore_axis_name="s")

    @functools.partial(
        pl.kernel,
        mesh=mesh,
        compiler_params=pltpu.CompilerParams(
            use_tc_tiling_on_sc=False, needs_layout_passes=False),
        out_type=jax.ShapeDtypeStruct((B, D), jnp.float32),
        scratch_types=[
            pltpu.VMEM((RPW * L,), jnp.int32),
            pltpu.VMEM((2 * GRP, L, W), jnp.uint32),
            pltpu.VMEM((RPW, D), jnp.float32),
            pltpu.SemaphoreType.DMA,
            pltpu.SemaphoreType.DMA,
        ],
    )
    def sc_pool(x_hbm, emb_hbm, out_hbm, idx_v, rows_v, out_v, sem0, sem1):
        wid = lax.axis_index("s") * NC + lax.axis_index("c")
        base = wid * RPW
        pltpu.sync_copy(x_hbm.at[pl.ds(base * L, RPW * L)], idx_v)

        def remap_body(i, carry):
            v = idx_v[pl.ds(i * LANES, LANES)]
            k = ((v >= Q).astype(jnp.int32)
                 + (v >= 2 * Q).astype(jnp.int32)
                 + (v >= 3 * Q).astype(jnp.int32))
            idx_v[pl.ds(i * LANES, LANES)] = 4 * v - (4 * Q - 1) * k
            return carry

        lax.fori_loop(0, NIV, remap_body, 0, unroll=8)

        def fire(g, half, sem):
            for j in range(GRP):
                row = g * GRP + j
                slot = half * GRP + j
                pltpu.async_copy(
                    emb_hbm.at[idx_v.at[pl.ds(row * L, CH0)]],
                    rows_v.at[slot, pl.ds(0, CH0)], sem)
                if CH1:
                    pltpu.async_copy(
                        emb_hbm.at[idx_v.at[pl.ds(row * L + CH0, CH1)]],
                        rows_v.at[slot, pl.ds(CH0, CH1)], sem)

        def drain(g, half, sem):
            for j in range(GRP):
                row = g * GRP + j
                slot = half * GRP + j
                pltpu.make_async_copy(
                    emb_hbm.at[idx_v.at[pl.ds(row * L, CH0)]],
                    rows_v.at[slot, pl.ds(0, CH0)], sem).wait()
                if CH1:
                    pltpu.make_async_copy(
                        emb_hbm.at[idx_v.at[pl.ds(row * L + CH0, CH1)]],
                        rows_v.at[slot, pl.ds(CH0, CH1)], sem).wait()

        def unpack(v):
            lo = plsc.bitcast(v << 16, jnp.float32)
            hi = plsc.bitcast(v & jnp.uint32(0xFFFF0000), jnp.float32)
            return lo, hi

        def compute(g, half):
            for j in range(GRP):
                row = g * GRP + j
                buf = rows_v.at[half * GRP + j]
                accs = []
                for c in range(nch):
                    lo, hi = unpack(buf[0, pl.ds(c * LANES, LANES)])
                    accs.extend((lo, hi))
                accs = tuple(accs)

                def max_body(r, accs):
                    out = []
                    for c in range(nch):
                        lo, hi = unpack(buf[r, pl.ds(c * LANES, LANES)])
                        out.append(jnp.maximum(accs[2 * c], lo))
                        out.append(jnp.maximum(accs[2 * c + 1], hi))
                    return tuple(out)

                accs = lax.fori_loop(1, L, max_body, accs, unroll=8)
                for c in range(nch):
                    out_v[row, pl.ds(c * LANES, LANES)] = accs[2 * c]
                    out_v[row, pl.ds((nch + c) * LANES, LANES)] = accs[2 * c + 1]

        fire(0, 0, sem0)

        def body(h, carry):
            g = 2 * h
            fire(g + 1, 1, sem1)
            drain(g, 0, sem0)
            compute(g, 0)

            @pl.when(h < NG // 2 - 1)
            def _():
                fire(g + 2, 0, sem0)

            drain(g + 1, 1, sem1)
            compute(g + 1, 1)
            return carry

        lax.fori_loop(0, NG // 2, body, 0)
        pltpu.sync_copy(out_v, out_hbm.at[pl.ds(base, RPW)])

    return sc_pool


@functools.lru_cache(maxsize=None)
def _make_tc_linear(B, D, C):
    BLK = min(B, 512)
    assert B % BLK == 0

    def body(p_ref, w_ref, b_ref, o_ref):
        acc = jnp.dot(p_ref[...], w_ref[...], preferred_element_type=jnp.float32)
        o_ref[...] = jax.nn.sigmoid(acc + b_ref[...])

    return pl.pallas_call(
        body,
        grid=(B // BLK,),
        in_specs=[
            pl.BlockSpec((BLK, D), lambda i: (i, 0)),
            pl.BlockSpec((D, C), lambda i: (0, 0)),
            pl.BlockSpec((1, C), lambda i: (0, 0)),
        ],
        out_specs=pl.BlockSpec((BLK, C), lambda i: (i, 0)),
        out_shape=jax.ShapeDtypeStruct((B, C), jnp.float32),
    )


def kernel(x, emb, fc_w, fc_b):
    B, L = x.shape
    V, D = emb.shape
    C = fc_w.shape[0]
    x = x.astype(jnp.int32)
    embT = jnp.swapaxes(emb, 0, 1)
    tblq = _make_tc_relayout(V, D)(embT, embT, embT, embT)
    tbl = jnp.reshape(tblq, (tblq.shape[0] * 4, D // 2))
    xflat = jnp.reshape(x, (B * L,))
    pooled = _make_sc_pool(B, L, D, V)(xflat, tbl)
    out = _make_tc_linear(B, D, C)(pooled, fc_w.T, fc_b.reshape(1, C))
    return out


# speedup vs baseline: 2.4078x; 1.0043x over previous
import functools

import jax
import numpy as np
import jax.numpy as jnp
from jax import lax
from jax.experimental import pallas as pl
from jax.experimental.pallas import tpu as pltpu
from jax.experimental.pallas import tpu_sc as plsc


LANES = 16
GRP = 2


def _split_point(V):
    BP = 2048
    nq = -(-((V + 3) // 4) // BP)
    return BP * nq, BP, nq


@functools.lru_cache(maxsize=None)
def _make_tc_relayout(V, D):
    Q, BP, nq = _split_point(V)
    last = (V - 1) // BP
    W = D // 2

    def body(a0, a1, a2, a3, o_ref):
        for k, a in enumerate((a0, a1, a2, a3)):
            u = jax.lax.bitcast_convert_type(a[...], jnp.uint32)
            lo = (u[0:W, :] + jnp.uint32(0x8000)) >> 16
            hi = (u[W:2 * W, :] + jnp.uint32(0x8000)) & jnp.uint32(0xFFFF0000)
            o_ref[:, k * W:(k + 1) * W] = (hi | lo).T

    def imap(k):
        return lambda j: (0, jnp.minimum(j + k * nq, last))

    return pl.pallas_call(
        body,
        grid=(nq,),
        in_specs=[pl.BlockSpec((D, BP), imap(k)) for k in range(4)],
        out_specs=pl.BlockSpec((BP, 4 * W), lambda j: (j, 0)),
        out_shape=jax.ShapeDtypeStruct((Q, 4 * W), jnp.uint32),
    )


@functools.lru_cache(maxsize=None)
def _make_sc_pool(B, L, D, V):
    Q = _split_point(V)[0]
    W = D // 2
    info = plsc.get_sparse_core_info()
    NC, NS = info.num_cores, info.num_subcores
    NW = NC * NS
    assert B % NW == 0
    RPW = B // NW
    assert RPW % (2 * GRP) == 0
    NG = RPW // GRP
    nch = W // LANES
    NIV = (RPW * L) // LANES

    CH0 = min(L, 128)
    CH0 -= CH0 % 8
    CH1 = L - CH0
    assert 0 < CH0 <= 128 and 0 <= CH1 <= 128 and CH0 % 8 == 0

    mesh = plsc.VectorSubcoreMesh(core_axis_name="c", subcore_axis_name="s")

    @functools.partial(
        pl.kernel,
        mesh=mesh,
        compiler_params=pltpu.CompilerParams(
            use_tc_tiling_on_sc=False, needs_layout_passes=False),
        out_type=jax.ShapeDtypeStruct((B, D), jnp.float32),
        scratch_types=[
            pltpu.VMEM((RPW * L,), jnp.int32),
            pltpu.VMEM((2 * GRP, L, W), jnp.uint32),
            pltpu.VMEM((RPW, D), jnp.float32),
            pltpu.SemaphoreType.DMA,
            pltpu.SemaphoreType.DMA,
        ],
    )
    def sc_pool(x_hbm, emb_hbm, out_hbm, idx_v, rows_v, out_v, sem0, sem1):
        wid = lax.axis_index("s") * NC + lax.axis_index("c")
        base = wid * RPW
        pltpu.sync_copy(x_hbm.at[pl.ds(base * L, RPW * L)], idx_v)

        def remap_body(i, carry):
            v = idx_v[pl.ds(i * LANES, LANES)]
            k = ((v >= Q).astype(jnp.int32)
                 + (v >= 2 * Q).astype(jnp.int32)
                 + (v >= 3 * Q).astype(jnp.int32))
            idx_v[pl.ds(i * LANES, LANES)] = 4 * v - (4 * Q - 1) * k
            return carry

        lax.fori_loop(0, NIV, remap_body, 0, unroll=8)

        def fire(g, half, sem):
            for j in range(GRP):
                row = g * GRP + j
                slot = half * GRP + j
                pltpu.async_copy(
                    emb_hbm.at[idx_v.at[pl.ds(row * L, CH0)]],
                    rows_v.at[slot, pl.ds(0, CH0)], sem)
                if CH1:
                    pltpu.async_copy(
                        emb_hbm.at[idx_v.at[pl.ds(row * L + CH0, CH1)]],
                        rows_v.at[slot, pl.ds(CH0, CH1)], sem)

        def drain(g, half, sem):
            for j in range(GRP):
                row = g * GRP + j
                slot = half * GRP + j
                pltpu.make_async_copy(
                    emb_hbm.at[idx_v.at[pl.ds(row * L, CH0)]],
                    rows_v.at[slot, pl.ds(0, CH0)], sem).wait()
                if CH1:
                    pltpu.make_async_copy(
                        emb_hbm.at[idx_v.at[pl.ds(row * L + CH0, CH1)]],
                        rows_v.at[slot, pl.ds(CH0, CH1)], sem).wait()

        def unpack(v):
            lo = plsc.bitcast(v << 16, jnp.float32)
            hi = plsc.bitcast(v & jnp.uint32(0xFFFF0000), jnp.float32)
            return lo, hi

        def compute(g, half):
            for j in range(GRP):
                row = g * GRP + j
                buf = rows_v.at[half * GRP + j]
                accs = []
                for c in range(nch):
                    lo, hi = unpack(buf[0, pl.ds(c * LANES, LANES)])
                    accs.extend((lo, hi))
                accs = tuple(accs)

                def max_body(r, accs):
                    out = []
                    for c in range(nch):
                        lo, hi = unpack(buf[r, pl.ds(c * LANES, LANES)])
                        out.append(jnp.maximum(accs[2 * c], lo))
                        out.append(jnp.maximum(accs[2 * c + 1], hi))
                    return tuple(out)

                accs = lax.fori_loop(1, L, max_body, accs, unroll=8)
                for c in range(nch):
                    out_v[row, pl.ds(c * LANES, LANES)] = accs[2 * c]
                    out_v[row, pl.ds((nch + c) * LANES, LANES)] = accs[2 * c + 1]

        fire(0, 0, sem0)

        def body(h, carry):
            g = 2 * h
            fire(g + 1, 1, sem1)
            drain(g, 0, sem0)
            compute(g, 0)

            @pl.when(h < NG // 2 - 1)
            def _():
                fire(g + 2, 0, sem0)

            drain(g + 1, 1, sem1)
            compute(g + 1, 1)
            return carry

        lax.fori_loop(0, NG // 2, body, 0)
        pltpu.sync_copy(out_v, out_hbm.at[pl.ds(base, RPW)])

    return sc_pool


@functools.lru_cache(maxsize=None)
def _make_tc_linear(B, D, C):
    BLK = min(B, 512)
    assert B % BLK == 0

    def body(p_ref, w_ref, b_ref, o_ref):
        acc = jnp.dot(p_ref[...], w_ref[...], preferred_element_type=jnp.float32)
        o_ref[...] = jax.nn.sigmoid(acc + b_ref[...])

    return pl.pallas_call(
        body,
        grid=(B // BLK,),
        in_specs=[
            pl.BlockSpec((BLK, D), lambda i: (i, 0)),
            pl.BlockSpec((D, C), lambda i: (0, 0)),
            pl.BlockSpec((1, C), lambda i: (0, 0)),
        ],
        out_specs=pl.BlockSpec((BLK, C), lambda i: (i, 0)),
        out_shape=jax.ShapeDtypeStruct((B, C), jnp.float32),
    )


def kernel(x, emb, fc_w, fc_b):
    B, L = x.shape
    V, D = emb.shape
    C = fc_w.shape[0]
    x = x.astype(jnp.int32)
    embT = jnp.swapaxes(emb, 0, 1)
    tblq = _make_tc_relayout(V, D)(embT, embT, embT, embT)
    tbl = jnp.reshape(tblq, (tblq.shape[0] * 4, D // 2))
    xflat = jnp.reshape(x, (B * L,))
    pooled = _make_sc_pool(B, L, D, V)(xflat, tbl)
    out = _make_tc_linear(B, D, C)(pooled, fc_w.T, fc_b.reshape(1, C))
    return out


# speedup vs baseline: 3.7830x; 1.5711x over previous
import functools

import jax
import numpy as np
import jax.numpy as jnp
from jax import lax
from jax.experimental import pallas as pl
from jax.experimental.pallas import tpu as pltpu
from jax.experimental.pallas import tpu_sc as plsc


LANES = 16
GRP = 2


def _split_point(V):
    BP = 4096
    nq = -(-((V + 3) // 4) // BP)
    return BP * nq, BP, nq


@functools.lru_cache(maxsize=None)
def _make_tc_relayout(V, D):
    Q, BP, nq = _split_point(V)
    last = (V - 1) // BP
    W = D // 2

    def body(a0, a1, a2, a3, o_ref):
        packs = []
        for a in (a0, a1, a2, a3):
            u = jax.lax.bitcast_convert_type(a[...], jnp.uint32)
            lo = (u[0:W, :] + jnp.uint32(0x8000)) >> 16
            hi = (u[W:2 * W, :] + jnp.uint32(0x8000)) & jnp.uint32(0xFFFF0000)
            packs.append(hi | lo)
        o_ref[...] = jnp.concatenate(packs, axis=0).T

    def imap(k):
        return lambda j: (0, jnp.minimum(j + k * nq, last))

    return pl.pallas_call(
        body,
        grid=(nq,),
        in_specs=[pl.BlockSpec((D, BP), imap(k)) for k in range(4)],
        out_specs=pl.BlockSpec((BP, 4 * W), lambda j: (j, 0)),
        out_shape=jax.ShapeDtypeStruct((Q, 4 * W), jnp.uint32),
    )


@functools.lru_cache(maxsize=None)
def _make_sc_pool(B, L, D, V):
    Q = _split_point(V)[0]
    W = D // 2
    info = plsc.get_sparse_core_info()
    NC, NS = info.num_cores, info.num_subcores
    NW = NC * NS
    assert B % NW == 0
    RPW = B // NW
    assert RPW % (2 * GRP) == 0
    NG = RPW // GRP
    nch = W // LANES
    NIV = (RPW * L) // LANES

    CH0 = min(L, 128)
    CH0 -= CH0 % 8
    CH1 = L - CH0
    assert 0 < CH0 <= 128 and 0 <= CH1 <= 128 and CH0 % 8 == 0

    mesh = plsc.VectorSubcoreMesh(core_axis_name="c", subcore_axis_name="s")

    @functools.partial(
        pl.kernel,
        mesh=mesh,
        compiler_params=pltpu.CompilerParams(
            use_tc_tiling_on_sc=False, needs_layout_passes=False),
        out_type=jax.ShapeDtypeStruct((B, D), jnp.float32),
        scratch_types=[
            pltpu.VMEM((RPW * L,), jnp.int32),
            pltpu.VMEM((2 * GRP, L, W), jnp.uint32),
            pltpu.VMEM((RPW, D), jnp.float32),
            pltpu.SemaphoreType.DMA,
            pltpu.SemaphoreType.DMA,
        ],
    )
    def sc_pool(x_hbm, emb_hbm, out_hbm, idx_v, rows_v, out_v, sem0, sem1):
        wid = lax.axis_index("s") * NC + lax.axis_index("c")
        base = wid * RPW
        pltpu.sync_copy(x_hbm.at[pl.ds(base * L, RPW * L)], idx_v)

        def remap_body(i, carry):
            v = idx_v[pl.ds(i * LANES, LANES)]
            k = ((v >= Q).astype(jnp.int32)
                 + (v >= 2 * Q).astype(jnp.int32)
                 + (v >= 3 * Q).astype(jnp.int32))
            idx_v[pl.ds(i * LANES, LANES)] = 4 * v - (4 * Q - 1) * k
            return carry

        lax.fori_loop(0, NIV, remap_body, 0, unroll=8)

        def fire(g, half, sem):
            for j in range(GRP):
                row = g * GRP + j
                slot = half * GRP + j
                pltpu.async_copy(
                    emb_hbm.at[idx_v.at[pl.ds(row * L, CH0)]],
                    rows_v.at[slot, pl.ds(0, CH0)], sem)
                if CH1:
                    pltpu.async_copy(
                        emb_hbm.at[idx_v.at[pl.ds(row * L + CH0, CH1)]],
                        rows_v.at[slot, pl.ds(CH0, CH1)], sem)

        def drain(g, half, sem):
            for j in range(GRP):
                row = g * GRP + j
                slot = half * GRP + j
                pltpu.make_async_copy(
                    emb_hbm.at[idx_v.at[pl.ds(row * L, CH0)]],
                    rows_v.at[slot, pl.ds(0, CH0)], sem).wait()
                if CH1:
                    pltpu.make_async_copy(
                        emb_hbm.at[idx_v.at[pl.ds(row * L + CH0, CH1)]],
                        rows_v.at[slot, pl.ds(CH0, CH1)], sem).wait()

        def unpack(v):
            lo = plsc.bitcast(v << 16, jnp.float32)
            hi = plsc.bitcast(v & jnp.uint32(0xFFFF0000), jnp.float32)
            return lo, hi

        def compute(g, half):
            for j in range(GRP):
                row = g * GRP + j
                buf = rows_v.at[half * GRP + j]
                accs = []
                for c in range(nch):
                    lo, hi = unpack(buf[0, pl.ds(c * LANES, LANES)])
                    accs.extend((lo, hi))
                accs = tuple(accs)

                def max_body(r, accs):
                    out = []
                    for c in range(nch):
                        lo, hi = unpack(buf[r, pl.ds(c * LANES, LANES)])
                        out.append(jnp.maximum(accs[2 * c], lo))
                        out.append(jnp.maximum(accs[2 * c + 1], hi))
                    return tuple(out)

                accs = lax.fori_loop(1, L, max_body, accs, unroll=8)
                for c in range(nch):
                    out_v[row, pl.ds(c * LANES, LANES)] = accs[2 * c]
                    out_v[row, pl.ds((nch + c) * LANES, LANES)] = accs[2 * c + 1]

        fire(0, 0, sem0)

        def body(h, carry):
            g = 2 * h
            fire(g + 1, 1, sem1)
            drain(g, 0, sem0)
            compute(g, 0)

            @pl.when(h < NG // 2 - 1)
            def _():
                fire(g + 2, 0, sem0)

            drain(g + 1, 1, sem1)
            compute(g + 1, 1)
            return carry

        lax.fori_loop(0, NG // 2, body, 0)
        pltpu.sync_copy(out_v, out_hbm.at[pl.ds(base, RPW)])

    return sc_pool


@functools.lru_cache(maxsize=None)
def _make_tc_linear(B, D, C):
    BLK = min(B, 512)
    assert B % BLK == 0

    def body(p_ref, w_ref, b_ref, o_ref):
        acc = jnp.dot(p_ref[...], w_ref[...], preferred_element_type=jnp.float32)
        o_ref[...] = jax.nn.sigmoid(acc + b_ref[...])

    return pl.pallas_call(
        body,
        grid=(B // BLK,),
        in_specs=[
            pl.BlockSpec((BLK, D), lambda i: (i, 0)),
            pl.BlockSpec((D, C), lambda i: (0, 0)),
            pl.BlockSpec((1, C), lambda i: (0, 0)),
        ],
        out_specs=pl.BlockSpec((BLK, C), lambda i: (i, 0)),
        out_shape=jax.ShapeDtypeStruct((B, C), jnp.float32),
    )


def kernel(x, emb, fc_w, fc_b):
    B, L = x.shape
    V, D = emb.shape
    C = fc_w.shape[0]
    x = x.astype(jnp.int32)
    embT = jnp.swapaxes(emb, 0, 1)
    tblq = _make_tc_relayout(V, D)(embT, embT, embT, embT)
    tbl = jnp.reshape(tblq, (tblq.shape[0] * 4, D // 2))
    xflat = jnp.reshape(x, (B * L,))
    pooled = _make_sc_pool(B, L, D, V)(xflat, tbl)
    out = _make_tc_linear(B, D, C)(pooled, fc_w.T, fc_b.reshape(1, C))
    return out


# speedup vs baseline: 3.9517x; 1.0446x over previous
import functools

import jax
import numpy as np
import jax.numpy as jnp
from jax import lax
from jax.experimental import pallas as pl
from jax.experimental.pallas import tpu as pltpu
from jax.experimental.pallas import tpu_sc as plsc


LANES = 16
GRP = 4


def _split_point(V):
    BP = 4096
    nq = -(-((V + 3) // 4) // BP)
    return BP * nq, BP, nq


@functools.lru_cache(maxsize=None)
def _make_tc_relayout(V, D):
    Q, BP, nq = _split_point(V)
    last = (V - 1) // BP
    W = D // 2

    def body(a0, a1, a2, a3, o_ref):
        packs = []
        for a in (a0, a1, a2, a3):
            u = jax.lax.bitcast_convert_type(a[...], jnp.uint32)
            lo = (u[0:W, :] + jnp.uint32(0x8000)) >> 16
            hi = (u[W:2 * W, :] + jnp.uint32(0x8000)) & jnp.uint32(0xFFFF0000)
            packs.append(hi | lo)
        o_ref[...] = jnp.concatenate(packs, axis=0).T

    def imap(k):
        return lambda j: (0, jnp.minimum(j + k * nq, last))

    return pl.pallas_call(
        body,
        grid=(nq,),
        in_specs=[pl.BlockSpec((D, BP), imap(k)) for k in range(4)],
        out_specs=pl.BlockSpec((BP, 4 * W), lambda j: (j, 0)),
        out_shape=jax.ShapeDtypeStruct((Q, 4 * W), jnp.uint32),
    )


@functools.lru_cache(maxsize=None)
def _make_sc_pool(B, L, D, V):
    Q = _split_point(V)[0]
    W = D // 2
    info = plsc.get_sparse_core_info()
    NC, NS = info.num_cores, info.num_subcores
    NW = NC * NS
    assert B % NW == 0
    RPW = B // NW
    assert RPW % (2 * GRP) == 0
    NG = RPW // GRP
    nch = W // LANES
    NIV = (RPW * L) // LANES

    CH0 = min(L, 128)
    CH0 -= CH0 % 8
    CH1 = L - CH0
    assert 0 < CH0 <= 128 and 0 <= CH1 <= 128 and CH0 % 8 == 0

    mesh = plsc.VectorSubcoreMesh(core_axis_name="c", subcore_axis_name="s")

    @functools.partial(
        pl.kernel,
        mesh=mesh,
        compiler_params=pltpu.CompilerParams(
            use_tc_tiling_on_sc=False, needs_layout_passes=False),
        out_type=jax.ShapeDtypeStruct((B, D), jnp.float32),
        scratch_types=[
            pltpu.VMEM((RPW * L,), jnp.int32),
            pltpu.VMEM((2 * GRP, L, W), jnp.uint32),
            pltpu.VMEM((RPW, D), jnp.float32),
            pltpu.SemaphoreType.DMA,
            pltpu.SemaphoreType.DMA,
        ],
    )
    def sc_pool(x_hbm, emb_hbm, out_hbm, idx_v, rows_v, out_v, sem0, sem1):
        wid = lax.axis_index("s") * NC + lax.axis_index("c")
        base = wid * RPW
        pltpu.sync_copy(x_hbm.at[pl.ds(base * L, RPW * L)], idx_v)

        def remap_body(i, carry):
            v = idx_v[pl.ds(i * LANES, LANES)]
            k = ((v >= Q).astype(jnp.int32)
                 + (v >= 2 * Q).astype(jnp.int32)
                 + (v >= 3 * Q).astype(jnp.int32))
            idx_v[pl.ds(i * LANES, LANES)] = 4 * v - (4 * Q - 1) * k
            return carry

        lax.fori_loop(0, NIV, remap_body, 0, unroll=8)

        def fire(g, half, sem):
            for j in range(GRP):
                row = g * GRP + j
                slot = half * GRP + j
                pltpu.async_copy(
                    emb_hbm.at[idx_v.at[pl.ds(row * L, CH0)]],
                    rows_v.at[slot, pl.ds(0, CH0)], sem)
                if CH1:
                    pltpu.async_copy(
                        emb_hbm.at[idx_v.at[pl.ds(row * L + CH0, CH1)]],
                        rows_v.at[slot, pl.ds(CH0, CH1)], sem)

        def drain(g, half, sem):
            for j in range(GRP):
                row = g * GRP + j
                slot = half * GRP + j
                pltpu.make_async_copy(
                    emb_hbm.at[idx_v.at[pl.ds(row * L, CH0)]],
                    rows_v.at[slot, pl.ds(0, CH0)], sem).wait()
                if CH1:
                    pltpu.make_async_copy(
                        emb_hbm.at[idx_v.at[pl.ds(row * L + CH0, CH1)]],
                        rows_v.at[slot, pl.ds(CH0, CH1)], sem).wait()

        def unpack(v):
            lo = plsc.bitcast(v << 16, jnp.float32)
            hi = plsc.bitcast(v & jnp.uint32(0xFFFF0000), jnp.float32)
            return lo, hi

        def compute(g, half):
            for j in range(GRP):
                row = g * GRP + j
                buf = rows_v.at[half * GRP + j]
                accs = []
                for c in range(nch):
                    lo, hi = unpack(buf[0, pl.ds(c * LANES, LANES)])
                    accs.extend((lo, hi))
                accs = tuple(accs)

                def max_body(r, accs):
                    out = []
                    for c in range(nch):
                        lo, hi = unpack(buf[r, pl.ds(c * LANES, LANES)])
                        out.append(jnp.maximum(accs[2 * c], lo))
                        out.append(jnp.maximum(accs[2 * c + 1], hi))
                    return tuple(out)

                accs = lax.fori_loop(1, L, max_body, accs, unroll=8)
                for c in range(nch):
                    out_v[row, pl.ds(c * LANES, LANES)] = accs[2 * c]
                    out_v[row, pl.ds((nch + c) * LANES, LANES)] = accs[2 * c + 1]

        fire(0, 0, sem0)

        def body(h, carry):
            g = 2 * h
            fire(g + 1, 1, sem1)
            drain(g, 0, sem0)
            compute(g, 0)

            @pl.when(h < NG // 2 - 1)
            def _():
                fire(g + 2, 0, sem0)

            drain(g + 1, 1, sem1)
            compute(g + 1, 1)
            return carry

        lax.fori_loop(0, NG // 2, body, 0)
        pltpu.sync_copy(out_v, out_hbm.at[pl.ds(base, RPW)])

    return sc_pool


@functools.lru_cache(maxsize=None)
def _make_tc_linear(B, D, C):
    BLK = min(B, 512)
    assert B % BLK == 0

    def body(p_ref, w_ref, b_ref, o_ref):
        acc = jnp.dot(p_ref[...], w_ref[...], preferred_element_type=jnp.float32)
        o_ref[...] = jax.nn.sigmoid(acc + b_ref[...])

    return pl.pallas_call(
        body,
        grid=(B // BLK,),
        in_specs=[
            pl.BlockSpec((BLK, D), lambda i: (i, 0)),
            pl.BlockSpec((D, C), lambda i: (0, 0)),
            pl.BlockSpec((1, C), lambda i: (0, 0)),
        ],
        out_specs=pl.BlockSpec((BLK, C), lambda i: (i, 0)),
        out_shape=jax.ShapeDtypeStruct((B, C), jnp.float32),
    )


def kernel(x, emb, fc_w, fc_b):
    B, L = x.shape
    V, D = emb.shape
    C = fc_w.shape[0]
    x = x.astype(jnp.int32)
    embT = jnp.swapaxes(emb, 0, 1)
    tblq = _make_tc_relayout(V, D)(embT, embT, embT, embT)
    tbl = jnp.reshape(tblq, (tblq.shape[0] * 4, D // 2))
    xflat = jnp.reshape(x, (B * L,))
    pooled = _make_sc_pool(B, L, D, V)(xflat, tbl)
    out = _make_tc_linear(B, D, C)(pooled, fc_w.T, fc_b.reshape(1, C))
    return out


# speedup vs baseline: 4.0367x; 1.0215x over previous
import functools

import jax
import numpy as np
import jax.numpy as jnp
from jax import lax
from jax.experimental import pallas as pl
from jax.experimental.pallas import tpu as pltpu
from jax.experimental.pallas import tpu_sc as plsc


LANES = 16
GRP = 4


def _split_point(V):
    BP = 8192
    nq = -(-((V + 3) // 4) // BP)
    return BP * nq, BP, nq


@functools.lru_cache(maxsize=None)
def _make_tc_relayout(V, D):
    Q, BP, nq = _split_point(V)
    last = (V - 1) // BP
    W = D // 2

    def body(a0, a1, a2, a3, o_ref):
        packs = []
        for a in (a0, a1, a2, a3):
            u = jax.lax.bitcast_convert_type(a[...], jnp.uint32)
            lo = (u[0:W, :] + jnp.uint32(0x8000)) >> 16
            hi = (u[W:2 * W, :] + jnp.uint32(0x8000)) & jnp.uint32(0xFFFF0000)
            packs.append(hi | lo)
        o_ref[...] = jnp.concatenate(packs, axis=0).T

    def imap(k):
        return lambda j: (0, jnp.minimum(j + k * nq, last))

    return pl.pallas_call(
        body,
        grid=(nq,),
        in_specs=[pl.BlockSpec((D, BP), imap(k)) for k in range(4)],
        out_specs=pl.BlockSpec((BP, 4 * W), lambda j: (j, 0)),
        out_shape=jax.ShapeDtypeStruct((Q, 4 * W), jnp.uint32),
    )


@functools.lru_cache(maxsize=None)
def _make_sc_pool(B, L, D, V):
    Q = _split_point(V)[0]
    W = D // 2
    info = plsc.get_sparse_core_info()
    NC, NS = info.num_cores, info.num_subcores
    NW = NC * NS
    assert B % NW == 0
    RPW = B // NW
    assert RPW % (2 * GRP) == 0
    NG = RPW // GRP
    nch = W // LANES
    NIV = (RPW * L) // LANES

    CH0 = min(L, 128)
    CH0 -= CH0 % 8
    CH1 = L - CH0
    assert 0 < CH0 <= 128 and 0 <= CH1 <= 128 and CH0 % 8 == 0

    mesh = plsc.VectorSubcoreMesh(core_axis_name="c", subcore_axis_name="s")

    @functools.partial(
        pl.kernel,
        mesh=mesh,
        compiler_params=pltpu.CompilerParams(
            use_tc_tiling_on_sc=False, needs_layout_passes=False),
        out_type=jax.ShapeDtypeStruct((B, D), jnp.float32),
        scratch_types=[
            pltpu.VMEM((RPW * L,), jnp.int32),
            pltpu.VMEM((2 * GRP, L, W), jnp.uint32),
            pltpu.VMEM((RPW, D), jnp.float32),
            pltpu.SemaphoreType.DMA,
            pltpu.SemaphoreType.DMA,
        ],
    )
    def sc_pool(x_hbm, emb_hbm, out_hbm, idx_v, rows_v, out_v, sem0, sem1):
        wid = lax.axis_index("s") * NC + lax.axis_index("c")
        base = wid * RPW
        pltpu.sync_copy(x_hbm.at[pl.ds(base * L, RPW * L)], idx_v)

        def remap_body(i, carry):
            v = idx_v[pl.ds(i * LANES, LANES)]
            k = ((v >= Q).astype(jnp.int32)
                 + (v >= 2 * Q).astype(jnp.int32)
                 + (v >= 3 * Q).astype(jnp.int32))
            idx_v[pl.ds(i * LANES, LANES)] = 4 * v - (4 * Q - 1) * k
            return carry

        lax.fori_loop(0, NIV, remap_body, 0, unroll=8)

        def fire(g, half, sem):
            for j in range(GRP):
                row = g * GRP + j
                slot = half * GRP + j
                pltpu.async_copy(
                    emb_hbm.at[idx_v.at[pl.ds(row * L, CH0)]],
                    rows_v.at[slot, pl.ds(0, CH0)], sem)
                if CH1:
                    pltpu.async_copy(
                        emb_hbm.at[idx_v.at[pl.ds(row * L + CH0, CH1)]],
                        rows_v.at[slot, pl.ds(CH0, CH1)], sem)

        def drain(g, half, sem):
            for j in range(GRP):
                row = g * GRP + j
                slot = half * GRP + j
                pltpu.make_async_copy(
                    emb_hbm.at[idx_v.at[pl.ds(row * L, CH0)]],
                    rows_v.at[slot, pl.ds(0, CH0)], sem).wait()
                if CH1:
                    pltpu.make_async_copy(
                        emb_hbm.at[idx_v.at[pl.ds(row * L + CH0, CH1)]],
                        rows_v.at[slot, pl.ds(CH0, CH1)], sem).wait()

        def unpack(v):
            lo = plsc.bitcast(v << 16, jnp.float32)
            hi = plsc.bitcast(v & jnp.uint32(0xFFFF0000), jnp.float32)
            return lo, hi

        def compute(g, half):
            for j in range(GRP):
                row = g * GRP + j
                buf = rows_v.at[half * GRP + j]
                accs = []
                for c in range(nch):
                    lo, hi = unpack(buf[0, pl.ds(c * LANES, LANES)])
                    accs.extend((lo, hi))
                accs = tuple(accs)

                def max_body(r, accs):
                    out = []
                    for c in range(nch):
                        lo, hi = unpack(buf[r, pl.ds(c * LANES, LANES)])
                        out.append(jnp.maximum(accs[2 * c], lo))
                        out.append(jnp.maximum(accs[2 * c + 1], hi))
                    return tuple(out)

                accs = lax.fori_loop(1, L, max_body, accs, unroll=8)
                for c in range(nch):
                    out_v[row, pl.ds(c * LANES, LANES)] = accs[2 * c]
                    out_v[row, pl.ds((nch + c) * LANES, LANES)] = accs[2 * c + 1]

        fire(0, 0, sem0)

        def body(h, carry):
            g = 2 * h
            fire(g + 1, 1, sem1)
            drain(g, 0, sem0)
            compute(g, 0)

            @pl.when(h < NG // 2 - 1)
            def _():
                fire(g + 2, 0, sem0)

            drain(g + 1, 1, sem1)
            compute(g + 1, 1)
            return carry

        lax.fori_loop(0, NG // 2, body, 0)
        pltpu.sync_copy(out_v, out_hbm.at[pl.ds(base, RPW)])

    return sc_pool


@functools.lru_cache(maxsize=None)
def _make_tc_linear(B, D, C):
    BLK = min(B, 512)
    assert B % BLK == 0

    def body(p_ref, w_ref, b_ref, o_ref):
        acc = jnp.dot(p_ref[...], w_ref[...], preferred_element_type=jnp.float32)
        o_ref[...] = jax.nn.sigmoid(acc + b_ref[...])

    return pl.pallas_call(
        body,
        grid=(B // BLK,),
        in_specs=[
            pl.BlockSpec((BLK, D), lambda i: (i, 0)),
            pl.BlockSpec((D, C), lambda i: (0, 0)),
            pl.BlockSpec((1, C), lambda i: (0, 0)),
        ],
        out_specs=pl.BlockSpec((BLK, C), lambda i: (i, 0)),
        out_shape=jax.ShapeDtypeStruct((B, C), jnp.float32),
    )


def kernel(x, emb, fc_w, fc_b):
    B, L = x.shape
    V, D = emb.shape
    C = fc_w.shape[0]
    x = x.astype(jnp.int32)
    embT = jnp.swapaxes(emb, 0, 1)
    tblq = _make_tc_relayout(V, D)(embT, embT, embT, embT)
    tbl = jnp.reshape(tblq, (tblq.shape[0] * 4, D // 2))
    xflat = jnp.reshape(x, (B * L,))
    pooled = _make_sc_pool(B, L, D, V)(xflat, tbl)
    out = _make_tc_linear(B, D, C)(pooled, fc_w.T, fc_b.reshape(1, C))
    return out
